# baseline (speedup 1.0000x reference)
import functools

import jax
import jax.numpy as jnp
from jax import lax
from jax.experimental import pallas as pl
from jax.experimental.pallas import tpu as pltpu
from jax.experimental.pallas import tpu_sc as plsc

_N = 1048576
_C = 16
_H = 16384
_W = 256
_NC = 2
_NS = 16
_NW = _NC * _NS
_PPW = _N // _NW
_K = 128
_NCHUNK = _PPW // _K
_L = 16


def _body(x0_hbm, x1_hbm, x2_hbm, table_hbm, out_hbm,
          x0v, x1v, x2v,
          i00, i01, i10, i11,
          w00, w01, w10, w11,
          r00, r01, r10, r11,
          outv, sem):
  wid = lax.axis_index("s") * _NC + lax.axis_index("c")
  base = wid * _PPW

  def chunk(c, carry):
    gb = base + c * _K
    pltpu.sync_copy(x0_hbm.at[pl.ds(gb, _K)], x0v)
    pltpu.sync_copy(x1_hbm.at[pl.ds(gb, _K)], x1v)
    pltpu.sync_copy(x2_hbm.at[pl.ds(gb, _K)], x2v)

    for i in range(_K // _L):
      s = i * _L
      sl = pl.ds(s, _L)
      a0 = x0v[sl]
      a1 = x1v[sl]
      a2 = x2v[sl]

      one = jnp.float32(1.0)
      zero = jnp.float32(0.0)
      c0 = jnp.minimum(jnp.maximum(a0, zero), one)
      c1 = jnp.minimum(jnp.maximum(a1, zero), one)
      c2 = jnp.minimum(jnp.maximum(a2, zero), one)
      three = jnp.int32(3)
      v0 = jnp.minimum((c0 * 4.0).astype(jnp.int32), three)
      v1 = jnp.minimum((c1 * 4.0).astype(jnp.int32), three)
      v2 = jnp.minimum((c2 * 4.0).astype(jnp.int32), three)
      vi = v2 * 16 + v1 * 4 + v0

      r0 = a0 * 1024.0
      r1 = a1 * 1024.0
      px = r0 - (r0 * (1.0 / 256.0)).astype(jnp.int32).astype(jnp.float32) * 256.0
      py = r1 - (r1 * (1.0 / 256.0)).astype(jnp.int32).astype(jnp.float32) * 256.0

      ix = px - 0.5
      iy = (py + vi.astype(jnp.float32) * 256.0) - 0.5
      ix0 = (ix + 1.0).astype(jnp.int32) - 1
      iy0 = (iy + 1.0).astype(jnp.int32) - 1
      wx1 = ix - ix0.astype(jnp.float32)
      wx0 = 1.0 - wx1
      wy1 = iy - iy0.astype(jnp.float32)
      wy0 = 1.0 - wy1

      inr = ((a0 >= 0.0) & (a0 <= 1.0) & (a1 >= 0.0) & (a1 <= 1.0))
      inrf = jnp.where(inr, one, zero)

      ix1 = ix0 + 1
      iy1 = iy0 + 1
      vx0 = jnp.where((ix0 >= 0) & (ix0 <= _W - 1), one, zero)
      vx1 = jnp.where((ix1 >= 0) & (ix1 <= _W - 1), one, zero)
      vy0 = jnp.where((iy0 >= 0) & (iy0 <= _H - 1), inrf, zero)
      vy1 = jnp.where((iy1 >= 0) & (iy1 <= _H - 1), inrf, zero)
      wx0 = wx0 * vx0
      wx1 = wx1 * vx1
      wy0 = wy0 * vy0
      wy1 = wy1 * vy1

      zi = jnp.int32(0)
      x0c = jnp.minimum(jnp.maximum(ix0, zi), jnp.int32(_W - 1))
      x1c = jnp.minimum(jnp.maximum(ix1, zi), jnp.int32(_W - 1))
      y0c = jnp.minimum(jnp.maximum(iy0, zi), jnp.int32(_H - 1)) * _W
      y1c = jnp.minimum(jnp.maximum(iy1, zi), jnp.int32(_H - 1)) * _W

      i00[sl] = y0c + x0c
      i01[sl] = y0c + x1c
      i10[sl] = y1c + x0c
      i11[sl] = y1c + x1c
      w00[sl] = wx0 * wy0
      w01[sl] = wx1 * wy0
      w10[sl] = wx0 * wy1
      w11[sl] = wx1 * wy1

    cp0 = pltpu.async_copy(table_hbm.at[i00], r00, sem)
    cp1 = pltpu.async_copy(table_hbm.at[i01], r01, sem)
    cp2 = pltpu.async_copy(table_hbm.at[i10], r10, sem)
    cp3 = pltpu.async_copy(table_hbm.at[i11], r11, sem)
    cp0.wait()
    cp1.wait()
    cp2.wait()
    cp3.wait()

    def acc(k, carry2):
      o = (r00[k] * w00[pl.ds(k, _L)][0]
           + r01[k] * w01[pl.ds(k, _L)][0]
           + r10[k] * w10[pl.ds(k, _L)][0]
           + r11[k] * w11[pl.ds(k, _L)][0])
      outv[k] = o
      return carry2

    lax.fori_loop(0, _K, acc, 0, unroll=4)

    pltpu.sync_copy(outv, out_hbm.at[pl.ds(gb, _K)])
    return carry

  lax.fori_loop(0, _NCHUNK, chunk, 0)


@jax.jit
def _run(x0, x1, x2, table):
  mesh = plsc.VectorSubcoreMesh(core_axis_name="c", subcore_axis_name="s")
  return pl.kernel(
      _body,
      out_type=jax.ShapeDtypeStruct((_N, _C), jnp.float32),
      mesh=mesh,
      scratch_types=[
          pltpu.VMEM((_K,), jnp.float32),
          pltpu.VMEM((_K,), jnp.float32),
          pltpu.VMEM((_K,), jnp.float32),
          pltpu.VMEM((_K,), jnp.int32),
          pltpu.VMEM((_K,), jnp.int32),
          pltpu.VMEM((_K,), jnp.int32),
          pltpu.VMEM((_K,), jnp.int32),
          pltpu.VMEM((_K + _L,), jnp.float32),
          pltpu.VMEM((_K + _L,), jnp.float32),
          pltpu.VMEM((_K + _L,), jnp.float32),
          pltpu.VMEM((_K + _L,), jnp.float32),
          pltpu.VMEM((_K, _C), jnp.float32),
          pltpu.VMEM((_K, _C), jnp.float32),
          pltpu.VMEM((_K, _C), jnp.float32),
          pltpu.VMEM((_K, _C), jnp.float32),
          pltpu.VMEM((_K, _C), jnp.float32),
          pltpu.SemaphoreType.DMA,
      ],
      compiler_params=pltpu.CompilerParams(use_tc_tiling_on_sc=False),
  )(x0, x1, x2, table)


_YB = 64


_G = _H // _YB
_XB = _N * 3 // _G // 128
_PB = _N // _G // 128


def _prep_body(t_ref, x_ref, tab_ref, x0_ref, x1_ref, x2_ref):
  b = t_ref[...].reshape(_C, _YB * _W)
  eye = jnp.eye(_C, dtype=jnp.float32)
  tab_ref[...] = jax.lax.dot_general(b, eye, (((0,), (0,)), ((), ())))
  X = x_ref[...].reshape(_PB, 3, 128)
  X0 = X[:, 0, :]
  X1 = X[:, 1, :]
  X2 = X[:, 2, :]
  ci = lax.broadcasted_iota(jnp.int32, (128, 128), 0)
  ji = lax.broadcasted_iota(jnp.int32, (128, 128), 1)
  for o, ref in ((0, x0_ref), (1, x1_ref), (2, x2_ref)):
    t = 3 * ji + o
    p0 = (t == ci).astype(jnp.float32)
    p1 = ((t - 128) == ci).astype(jnp.float32)
    p2 = ((t - 256) == ci).astype(jnp.float32)
    ref[...] = X0 @ p0 + X1 @ p1 + X2 @ p2


@jax.jit
def _prep(tensor, xr):
  return pl.pallas_call(
      _prep_body,
      grid=(_G,),
      in_specs=[
          pl.BlockSpec((_C, _YB, _W), lambda i: (0, i, 0)),
          pl.BlockSpec((_XB, 128), lambda i: (i, 0)),
      ],
      out_specs=[
          pl.BlockSpec((_YB * _W, _C), lambda i: (i, 0)),
          pl.BlockSpec((_PB, 128), lambda i: (i, 0)),
          pl.BlockSpec((_PB, 128), lambda i: (i, 0)),
          pl.BlockSpec((_PB, 128), lambda i: (i, 0)),
      ],
      out_shape=[
          jax.ShapeDtypeStruct((_H * _W, _C), jnp.float32),
          jax.ShapeDtypeStruct((_N // 128, 128), jnp.float32),
          jax.ShapeDtypeStruct((_N // 128, 128), jnp.float32),
          jax.ShapeDtypeStruct((_N // 128, 128), jnp.float32),
      ],
  )(tensor, xr)


def kernel(x, tensor):
  table, x0, x1, x2 = _prep(tensor, x.reshape(_N * 3 // 128, 128))
  return _run(x0.reshape(_N), x1.reshape(_N), x2.reshape(_N), table)


# speedup vs baseline: 1.5010x; 1.5010x over previous
import functools

import jax
import jax.numpy as jnp
from jax import lax
from jax.experimental import pallas as pl
from jax.experimental.pallas import tpu as pltpu
from jax.experimental.pallas import tpu_sc as plsc

_N = 1048576
_C = 16
_H = 16384
_W = 256
_NC = 2
_NS = 16
_NW = _NC * _NS
_PPW = _N // _NW
_K = 128
_NCHUNK = _PPW // _K
_L = 16


def _compute_indices(x0v, x1v, x2v, i00, i01, i10, i11, w00, w01, w10, w11):
  for i in range(_K // _L):
    s = i * _L
    sl = pl.ds(s, _L)
    a0 = x0v[sl]
    a1 = x1v[sl]
    a2 = x2v[sl]

    one = jnp.float32(1.0)
    zero = jnp.float32(0.0)
    c0 = jnp.minimum(jnp.maximum(a0, zero), one)
    c1 = jnp.minimum(jnp.maximum(a1, zero), one)
    c2 = jnp.minimum(jnp.maximum(a2, zero), one)
    three = jnp.int32(3)
    v0 = jnp.minimum((c0 * 4.0).astype(jnp.int32), three)
    v1 = jnp.minimum((c1 * 4.0).astype(jnp.int32), three)
    v2 = jnp.minimum((c2 * 4.0).astype(jnp.int32), three)
    vi = v2 * 16 + v1 * 4 + v0

    r0 = a0 * 1024.0
    r1 = a1 * 1024.0
    px = r0 - (r0 * (1.0 / 256.0)).astype(jnp.int32).astype(jnp.float32) * 256.0
    py = r1 - (r1 * (1.0 / 256.0)).astype(jnp.int32).astype(jnp.float32) * 256.0

    ix = px - 0.5
    iy = (py + vi.astype(jnp.float32) * 256.0) - 0.5
    ix0 = (ix + 1.0).astype(jnp.int32) - 1
    iy0 = (iy + 1.0).astype(jnp.int32) - 1
    wx1 = ix - ix0.astype(jnp.float32)
    wx0 = 1.0 - wx1
    wy1 = iy - iy0.astype(jnp.float32)
    wy0 = 1.0 - wy1

    inr = ((a0 >= 0.0) & (a0 <= 1.0) & (a1 >= 0.0) & (a1 <= 1.0))
    inrf = jnp.where(inr, one, zero)

    ix1 = ix0 + 1
    iy1 = iy0 + 1
    vx0 = jnp.where((ix0 >= 0) & (ix0 <= _W - 1), one, zero)
    vx1 = jnp.where((ix1 >= 0) & (ix1 <= _W - 1), one, zero)
    vy0 = jnp.where((iy0 >= 0) & (iy0 <= _H - 1), inrf, zero)
    vy1 = jnp.where((iy1 >= 0) & (iy1 <= _H - 1), inrf, zero)
    wx0 = wx0 * vx0
    wx1 = wx1 * vx1
    wy0 = wy0 * vy0
    wy1 = wy1 * vy1

    zi = jnp.int32(0)
    x0c = jnp.minimum(jnp.maximum(ix0, zi), jnp.int32(_W - 1))
    x1c = jnp.minimum(jnp.maximum(ix1, zi), jnp.int32(_W - 1))
    y0c = jnp.minimum(jnp.maximum(iy0, zi), jnp.int32(_H - 1)) * _W
    y1c = jnp.minimum(jnp.maximum(iy1, zi), jnp.int32(_H - 1)) * _W

    i00[sl] = y0c + x0c
    i01[sl] = y0c + x1c
    i10[sl] = y1c + x0c
    i11[sl] = y1c + x1c
    w00[sl] = wx0 * wy0
    w01[sl] = wx1 * wy0
    w10[sl] = wx0 * wy1
    w11[sl] = wx1 * wy1


def _accumulate(r00, r01, r10, r11, w00, w01, w10, w11, outv):
  def acc(k, carry2):
    o = (r00[k] * w00[pl.ds(k, _L)][0]
         + r01[k] * w01[pl.ds(k, _L)][0]
         + r10[k] * w10[pl.ds(k, _L)][0]
         + r11[k] * w11[pl.ds(k, _L)][0])
    outv[k] = o
    return carry2

  lax.fori_loop(0, _K, acc, 0, unroll=4)


def _body(x0_hbm, x1_hbm, x2_hbm, table_hbm, out_hbm,
          x0v, x1v, x2v,
          i00a, i01a, i10a, i11a, w00a, w01a, w10a, w11a,
          r00a, r01a, r10a, r11a,
          i00b, i01b, i10b, i11b, w00b, w01b, w10b, w11b,
          r00b, r01b, r10b, r11b,
          outv, sema, semb):
  wid = lax.axis_index("s") * _NC + lax.axis_index("c")
  base = wid * _PPW

  def _load_coords(gb):
    pltpu.sync_copy(x0_hbm.at[pl.ds(gb, _K)], x0v)
    pltpu.sync_copy(x1_hbm.at[pl.ds(gb, _K)], x1v)
    pltpu.sync_copy(x2_hbm.at[pl.ds(gb, _K)], x2v)

  def chunk2(c2, carry):
    gba = base + (2 * c2) * _K
    gbb = gba + _K

    _load_coords(gba)
    _compute_indices(x0v, x1v, x2v, i00a, i01a, i10a, i11a,
                     w00a, w01a, w10a, w11a)
    cpa = [pltpu.async_copy(table_hbm.at[i00a], r00a, sema),
           pltpu.async_copy(table_hbm.at[i01a], r01a, sema),
           pltpu.async_copy(table_hbm.at[i10a], r10a, sema),
           pltpu.async_copy(table_hbm.at[i11a], r11a, sema)]

    _load_coords(gbb)
    _compute_indices(x0v, x1v, x2v, i00b, i01b, i10b, i11b,
                     w00b, w01b, w10b, w11b)
    cpb = [pltpu.async_copy(table_hbm.at[i00b], r00b, semb),
           pltpu.async_copy(table_hbm.at[i01b], r01b, semb),
           pltpu.async_copy(table_hbm.at[i10b], r10b, semb),
           pltpu.async_copy(table_hbm.at[i11b], r11b, semb)]

    for cp in cpa:
      cp.wait()
    _accumulate(r00a, r01a, r10a, r11a, w00a, w01a, w10a, w11a, outv)
    pltpu.sync_copy(outv, out_hbm.at[pl.ds(gba, _K)])

    for cp in cpb:
      cp.wait()
    _accumulate(r00b, r01b, r10b, r11b, w00b, w01b, w10b, w11b, outv)
    pltpu.sync_copy(outv, out_hbm.at[pl.ds(gbb, _K)])
    return carry

  lax.fori_loop(0, _NCHUNK // 2, chunk2, 0)


@jax.jit
def _run(x0, x1, x2, table):
  mesh = plsc.VectorSubcoreMesh(core_axis_name="c", subcore_axis_name="s")
  return pl.kernel(
      _body,
      out_type=jax.ShapeDtypeStruct((_N, _C), jnp.float32),
      mesh=mesh,
      scratch_types=(
          [pltpu.VMEM((_K,), jnp.float32)] * 3
          + ([pltpu.VMEM((_K,), jnp.int32)] * 4
             + [pltpu.VMEM((_K + _L,), jnp.float32)] * 4
             + [pltpu.VMEM((_K, _C), jnp.float32)] * 4) * 2
          + [pltpu.VMEM((_K, _C), jnp.float32)]
          + [pltpu.SemaphoreType.DMA] * 2
      ),
      compiler_params=pltpu.CompilerParams(use_tc_tiling_on_sc=False),
  )(x0, x1, x2, table)


_YB = 64


def _tpose_body(in_ref, out_ref):
  b = in_ref[...].reshape(_C, _YB * _W)
  eye = jnp.eye(_C, dtype=jnp.float32)
  out_ref[...] = jax.lax.dot_general(b, eye, (((0,), (0,)), ((), ())))


@jax.jit
def _detile(tensor):
  return pl.pallas_call(
      _tpose_body,
      grid=(_H // _YB,),
      in_specs=[pl.BlockSpec((_C, _YB, _W), lambda i: (0, i, 0))],
      out_specs=pl.BlockSpec((_YB * _W, _C), lambda i: (i, 0)),
      out_shape=jax.ShapeDtypeStruct((_H * _W, _C), jnp.float32),
  )(tensor)


def kernel(x, tensor):
  return _run(x[:, 0], x[:, 1], x[:, 2], _detile(tensor))


# speedup vs baseline: 1.6904x; 1.1262x over previous
import functools

import jax
import jax.numpy as jnp
from jax import lax
from jax.experimental import pallas as pl
from jax.experimental.pallas import tpu as pltpu
from jax.experimental.pallas import tpu_sc as plsc

_N = 1048576
_C = 16
_H = 16384
_W = 256
_NC = 2
_NS = 16
_NW = _NC * _NS
_PPW = _N // _NW
_K = 128
_NCHUNK = _PPW // _K
_L = 16


def _compute_indices(x0v, x1v, x2v, i00, i01, i10, i11, w00, w01, w10, w11):
  for i in range(_K // _L):
    s = i * _L
    sl = pl.ds(s, _L)
    a0 = x0v[sl]
    a1 = x1v[sl]
    a2 = x2v[sl]

    one = jnp.float32(1.0)
    zero = jnp.float32(0.0)
    c0 = jnp.minimum(jnp.maximum(a0, zero), one)
    c1 = jnp.minimum(jnp.maximum(a1, zero), one)
    c2 = jnp.minimum(jnp.maximum(a2, zero), one)
    three = jnp.int32(3)
    v0 = jnp.minimum((c0 * 4.0).astype(jnp.int32), three)
    v1 = jnp.minimum((c1 * 4.0).astype(jnp.int32), three)
    v2 = jnp.minimum((c2 * 4.0).astype(jnp.int32), three)
    vi = v2 * 16 + v1 * 4 + v0

    r0 = a0 * 1024.0
    r1 = a1 * 1024.0
    px = r0 - (r0 * (1.0 / 256.0)).astype(jnp.int32).astype(jnp.float32) * 256.0
    py = r1 - (r1 * (1.0 / 256.0)).astype(jnp.int32).astype(jnp.float32) * 256.0

    ix = px - 0.5
    iy = (py + vi.astype(jnp.float32) * 256.0) - 0.5
    ix0 = (ix + 1.0).astype(jnp.int32) - 1
    iy0 = (iy + 1.0).astype(jnp.int32) - 1
    wx1 = ix - ix0.astype(jnp.float32)
    wx0 = 1.0 - wx1
    wy1 = iy - iy0.astype(jnp.float32)
    wy0 = 1.0 - wy1

    inr = ((a0 >= 0.0) & (a0 <= 1.0) & (a1 >= 0.0) & (a1 <= 1.0))
    inrf = jnp.where(inr, one, zero)

    ix1 = ix0 + 1
    iy1 = iy0 + 1
    vx0 = jnp.where((ix0 >= 0) & (ix0 <= _W - 1), one, zero)
    vx1 = jnp.where((ix1 >= 0) & (ix1 <= _W - 1), one, zero)
    vy0 = jnp.where((iy0 >= 0) & (iy0 <= _H - 1), inrf, zero)
    vy1 = jnp.where((iy1 >= 0) & (iy1 <= _H - 1), inrf, zero)
    wx0 = wx0 * vx0
    wx1 = wx1 * vx1
    wy0 = wy0 * vy0
    wy1 = wy1 * vy1

    zi = jnp.int32(0)
    x0c = jnp.minimum(jnp.maximum(ix0, zi), jnp.int32(_W - 1))
    x1c = jnp.minimum(jnp.maximum(ix1, zi), jnp.int32(_W - 1))
    y0c = jnp.minimum(jnp.maximum(iy0, zi), jnp.int32(_H - 1)) * _W
    y1c = jnp.minimum(jnp.maximum(iy1, zi), jnp.int32(_H - 1)) * _W

    i00[sl] = y0c + x0c
    i01[sl] = y0c + x1c
    i10[sl] = y1c + x0c
    i11[sl] = y1c + x1c
    w00[sl] = wx0 * wy0
    w01[sl] = wx1 * wy0
    w10[sl] = wx0 * wy1
    w11[sl] = wx1 * wy1


def _accumulate(r00, r01, r10, r11, w00, w01, w10, w11, outv):
  def acc(k, carry2):
    o = (r00[k] * w00[pl.ds(k, _L)][0]
         + r01[k] * w01[pl.ds(k, _L)][0]
         + r10[k] * w10[pl.ds(k, _L)][0]
         + r11[k] * w11[pl.ds(k, _L)][0])
    outv[k] = o
    return carry2

  lax.fori_loop(0, _K, acc, 0, unroll=4)


def _body(x0_hbm, x1_hbm, x2_hbm, table_hbm, out_hbm,
          x0v, x1v, x2v,
          i00a, i01a, i10a, i11a, w00a, w01a, w10a, w11a,
          r00a, r01a, r10a, r11a,
          i00b, i01b, i10b, i11b, w00b, w01b, w10b, w11b,
          r00b, r01b, r10b, r11b,
          outv, sema, semb):
  wid = lax.axis_index("s") * _NC + lax.axis_index("c")
  base = wid * _PPW

  def _load_coords(gb):
    pltpu.sync_copy(x0_hbm.at[pl.ds(gb, _K)], x0v)
    pltpu.sync_copy(x1_hbm.at[pl.ds(gb, _K)], x1v)
    pltpu.sync_copy(x2_hbm.at[pl.ds(gb, _K)], x2v)

  def chunk2(c2, carry):
    gba = base + (2 * c2) * _K
    gbb = gba + _K

    _load_coords(gba)
    _compute_indices(x0v, x1v, x2v, i00a, i01a, i10a, i11a,
                     w00a, w01a, w10a, w11a)
    cpa = [pltpu.async_copy(table_hbm.at[i00a], r00a, sema),
           pltpu.async_copy(table_hbm.at[i01a], r01a, sema),
           pltpu.async_copy(table_hbm.at[i10a], r10a, sema),
           pltpu.async_copy(table_hbm.at[i11a], r11a, sema)]

    _load_coords(gbb)
    _compute_indices(x0v, x1v, x2v, i00b, i01b, i10b, i11b,
                     w00b, w01b, w10b, w11b)
    cpb = [pltpu.async_copy(table_hbm.at[i00b], r00b, semb),
           pltpu.async_copy(table_hbm.at[i01b], r01b, semb),
           pltpu.async_copy(table_hbm.at[i10b], r10b, semb),
           pltpu.async_copy(table_hbm.at[i11b], r11b, semb)]

    for cp in cpa:
      cp.wait()
    _accumulate(r00a, r01a, r10a, r11a, w00a, w01a, w10a, w11a, outv)
    pltpu.sync_copy(outv, out_hbm.at[pl.ds(gba, _K)])

    for cp in cpb:
      cp.wait()
    _accumulate(r00b, r01b, r10b, r11b, w00b, w01b, w10b, w11b, outv)
    pltpu.sync_copy(outv, out_hbm.at[pl.ds(gbb, _K)])
    return carry

  lax.fori_loop(0, _NCHUNK // 2, chunk2, 0)


@jax.jit
def _run(x0, x1, x2, table):
  mesh = plsc.VectorSubcoreMesh(core_axis_name="c", subcore_axis_name="s")
  return pl.kernel(
      _body,
      out_type=jax.ShapeDtypeStruct((_N, _C), jnp.float32),
      mesh=mesh,
      scratch_types=(
          [pltpu.VMEM((_K,), jnp.float32)] * 3
          + ([pltpu.VMEM((_K,), jnp.int32)] * 4
             + [pltpu.VMEM((_K + _L,), jnp.float32)] * 4
             + [pltpu.VMEM((_K, _C), jnp.float32)] * 4) * 2
          + [pltpu.VMEM((_K, _C), jnp.float32)]
          + [pltpu.SemaphoreType.DMA] * 2
      ),
      compiler_params=pltpu.CompilerParams(use_tc_tiling_on_sc=False),
  )(x0, x1, x2, table)


_YB = 64


def _tpose_body(in_ref, out_ref):
  m = _YB * _W
  b = in_ref[...].reshape(_C, m)
  eye = jnp.eye(_C, dtype=jnp.float32)
  t = jax.lax.dot_general(b, eye, (((0,), (0,)), ((), ())))
  ci = lax.broadcasted_iota(jnp.int32, (_C, 128), 0)
  li = lax.broadcasted_iota(jnp.int32, (_C, 128), 1)
  acc = jnp.zeros((m // 8, 128), jnp.float32)
  for p in range(8):
    tp = lax.slice(t, (p, 0), (p + 8 * (m // 8 - 1) + 1, _C), (8, 1))
    gp = (li == 16 * p + ci).astype(jnp.float32)
    acc = acc + tp @ gp
  out_ref[...] = acc


@jax.jit
def _detile(tensor):
  return pl.pallas_call(
      _tpose_body,
      grid=(_H // _YB,),
      in_specs=[pl.BlockSpec((_C, _YB, _W), lambda i: (0, i, 0))],
      out_specs=pl.BlockSpec((_YB * _W * _C // 128, 128), lambda i: (i, 0)),
      out_shape=jax.ShapeDtypeStruct((_H * _W * _C // 128, 128), jnp.float32),
  )(tensor)


def kernel(x, tensor):
  table = jnp.transpose(tensor, (1, 2, 0)).reshape(_H * _W, _C)
  return _run(x[:, 0], x[:, 1], x[:, 2], table)
